# baseline (speedup 1.0000x reference)
import jax
import jax.numpy as jnp
from jax import lax
from jax.experimental import pallas as pl
from jax.experimental.pallas import tpu as pltpu
from jax.experimental.pallas import tpu_sc as plsc

N = 10000
E = 160000
D = 128

NC = 2
NS = 16
L = 16
NW = NC * NS
EPW = 5120
E_PAD = NW * EPW
CH = 128
NCHUNK = EPW // CH
R = 10240
TRASH = N
RPT = R // NS
TSIZE = N * N + 128

ROW_BLOCK = 1000


def _support_body(x_ref, w_ref, o_ref):
    o_ref[...] = jnp.dot(x_ref[...], w_ref[...],
                         preferred_element_type=jnp.float32)


def _combine_body(p_ref, b_ref, o_ref):
    o_ref[...] = p_ref[0] + p_ref[1] + b_ref[...]


def _claim_body(skeys2d, tbl, kvm, vm, sem):
    cid = lax.axis_index("c")
    sid = lax.axis_index("s")
    wid = sid * NC + cid
    base = wid * EPW

    pltpu.sync_copy(skeys2d.at[pl.ds(wid * NCHUNK, NCHUNK)], kvm)
    lane = lax.iota(jnp.int32, L)
    for c in range(NCHUNK):
        for g in range(CH // L):
            vm[c, pl.ds(g * L, L)] = base + c * CH + g * L + lane
    cps = [pltpu.async_copy(vm.at[c], tbl.at[kvm.at[c]], sem)
           for c in range(NCHUNK)]
    for cp in cps:
        cp.wait()


def _agg_body(skeys2d, tbl, support, zrows, pout,
              kvm, tb, vidx, uidx, rows, acc, semt0, semt1, semr0, semr1):
    cid = lax.axis_index("c")
    sid = lax.axis_index("s")
    wid = sid * NC + cid
    semt = [semt0, semt1]
    semr = [semr0, semr1]

    pltpu.sync_copy(zrows, acc.at[pl.ds(sid * RPT, RPT)])

    pltpu.sync_copy(skeys2d.at[pl.ds(wid * NCHUNK, NCHUNK)], kvm)
    plsc.subcore_barrier()

    base = wid * EPW
    lane = lax.iota(jnp.int32, L)

    def _fire_claim_gather(k, b):
        pltpu.async_copy(tbl.at[kvm.at[k]], tb.at[b], semt[b])

    def _wait_claim_gather(k, b):
        pltpu.make_async_copy(tbl.at[kvm.at[k]], tb.at[b], semt[b]).wait()

    def _decode_fire_rowgather(k, b):
        for g in range(CH // L):
            cur = kvm[k, pl.ds(g * L, L)]
            win = tb[b, pl.ds(g * L, L)]
            eid = base + k * CH + g * L + lane
            u = lax.div(cur, N)
            v = cur - u * N
            ue = jnp.where(win == eid, u, TRASH)
            vidx[b, pl.ds(g * L, L)] = v
            uidx[b, pl.ds(g * L, L)] = ue
        pltpu.async_copy(support.at[vidx.at[b]],
                         rows.at[pl.ds(b * CH, CH)], semr[b])

    def _wait_rowgather(b):
        pltpu.make_async_copy(support.at[vidx.at[b]],
                              rows.at[pl.ds(b * CH, CH)], semr[b]).wait()

    def _scatter(b):
        pltpu.sync_copy(rows.at[pl.ds(b * CH, CH)],
                        acc.at[uidx.at[b]], add=True)

    _fire_claim_gather(0, 0)
    _wait_claim_gather(0, 0)
    _fire_claim_gather(1, 1)
    _decode_fire_rowgather(0, 0)

    @pl.loop(0, NCHUNK - 2, step=2)
    def _chunk(c):
        for b in range(2):
            k = c + b
            _wait_claim_gather(k + 1, 1 - b)
            _fire_claim_gather(k + 2, b)
            _decode_fire_rowgather(k + 1, 1 - b)
            _wait_rowgather(b)
            _scatter(b)

    _wait_claim_gather(NCHUNK - 1, 1)
    _decode_fire_rowgather(NCHUNK - 1, 1)
    _wait_rowgather(0)
    _scatter(0)
    _wait_rowgather(1)
    _scatter(1)

    plsc.subcore_barrier()
    pltpu.sync_copy(acc.at[pl.ds(sid * RPT, RPT)],
                    pout.at[cid, pl.ds(sid * RPT, RPT)])


def _make_claim():
    mesh = plsc.VectorSubcoreMesh(core_axis_name="c", subcore_axis_name="s",
                                  num_cores=NC, num_subcores=NS)
    return pl.kernel(
        _claim_body,
        out_type=jax.ShapeDtypeStruct((TSIZE,), jnp.int32),
        mesh=mesh,
        scratch_types=[
            pltpu.VMEM((NCHUNK, CH), jnp.int32),
            pltpu.VMEM((NCHUNK, CH), jnp.int32),
            pltpu.SemaphoreType.DMA,
        ],
    )


def _make_agg():
    mesh = plsc.VectorSubcoreMesh(core_axis_name="c", subcore_axis_name="s",
                                  num_cores=NC, num_subcores=NS)
    return pl.kernel(
        _agg_body,
        out_type=jax.ShapeDtypeStruct((NC, R, D), jnp.float32),
        mesh=mesh,
        scratch_types=[
            pltpu.VMEM((NCHUNK, CH), jnp.int32),
            pltpu.VMEM((2, CH), jnp.int32),
            pltpu.VMEM((2, CH), jnp.int32),
            pltpu.VMEM((2, CH), jnp.int32),
            pltpu.VMEM((2 * CH, D), jnp.float32),
            pltpu.VMEM_SHARED((R, D), jnp.float32),
            pltpu.SemaphoreType.DMA,
            pltpu.SemaphoreType.DMA,
            pltpu.SemaphoreType.DMA,
            pltpu.SemaphoreType.DMA,
        ],
    )


def kernel(input, edge_index, W, b):
    x = input
    n = x.shape[0]
    assert n == N and x.shape[1] == D and edge_index.shape == (2, E)

    support = pl.pallas_call(
        _support_body,
        grid=(N // ROW_BLOCK,),
        in_specs=[pl.BlockSpec((ROW_BLOCK, D), lambda i: (i, 0)),
                  pl.BlockSpec((D, D), lambda i: (0, 0))],
        out_specs=pl.BlockSpec((ROW_BLOCK, D), lambda i: (i, 0)),
        out_shape=jax.ShapeDtypeStruct((N, D), jnp.float32),
    )(x, W)

    enc = edge_index[0].astype(jnp.int32) * N + edge_index[1].astype(jnp.int32)
    skeys2d = jnp.concatenate([
        enc,
        jnp.full((E_PAD - E,), N * N, jnp.int32),
    ]).reshape(E_PAD // CH, CH)
    zrows = jnp.zeros((RPT, D), jnp.float32)

    tbl = _make_claim()(skeys2d)
    pout = _make_agg()(skeys2d, tbl, support, zrows)

    out = pl.pallas_call(
        _combine_body,
        grid=(N // ROW_BLOCK,),
        in_specs=[pl.BlockSpec((NC, ROW_BLOCK, D), lambda i: (0, i, 0)),
                  pl.BlockSpec((1, D), lambda i: (0, 0))],
        out_specs=pl.BlockSpec((ROW_BLOCK, D), lambda i: (i, 0)),
        out_shape=jax.ShapeDtypeStruct((N, D), jnp.float32),
    )(pout, b.reshape(1, D))
    return out


# speedup vs baseline: 1.9155x; 1.9155x over previous
import jax
import jax.numpy as jnp
from jax import lax
from jax.experimental import pallas as pl
from jax.experimental.pallas import tpu as pltpu
from jax.experimental.pallas import tpu_sc as plsc

N = 10000
E = 160000
D = 128

NC = 2
NS = 16
L = 16
NW = NC * NS
EPW = 5120
E_PAD = NW * EPW
CH = 128
NCHUNK = EPW // CH
R = 10240
TRASH = N
RPT = R // NS

ROW_BLOCK = 1000


def _support_body(x_ref, w_ref, o_ref):
    o_ref[...] = jnp.dot(x_ref[...], w_ref[...],
                         preferred_element_type=jnp.float32)


def _combine_body(p_ref, b_ref, o_ref):
    o_ref[...] = p_ref[0] + p_ref[1] + b_ref[...]


def _agg_body(skeys, support, zrows, pout, kv0, kv1, vidx, uidx, rows, acc,
              sem0, sem1):
    cid = lax.axis_index("c")
    sid = lax.axis_index("s")
    wid = sid * NC + cid
    sems = [sem0, sem1]
    kvs = [kv0, kv1]

    pltpu.sync_copy(zrows, acc.at[pl.ds(sid * RPT, RPT)])
    plsc.subcore_barrier()

    base = 8 + wid * EPW

    def _prep(k, b):
        goff = base + k * CH
        pltpu.sync_copy(skeys.at[pl.ds(goff - 8, CH + 8)], kvs[b])
        for i in range(CH // L):
            cur = kvs[b][pl.ds(8 + i * L, L)]
            prv = kvs[b][pl.ds(7 + i * L, L)]
            u = lax.div(cur, N)
            v = cur - u * N
            ue = jnp.where(cur == prv, TRASH, u)
            vidx[b, pl.ds(i * L, L)] = v
            uidx[b, pl.ds(i * L, L)] = ue
        for h in range(4):
            pltpu.async_copy(
                support.at[vidx.at[b, pl.ds(h * 32, 32)]],
                rows.at[pl.ds(b * CH + h * 32, 32)], sems[b])

    _prep(0, 0)

    @pl.loop(0, NCHUNK, step=2)
    def _chunk(c):
        for b in range(2):
            k = c + b
            for h in range(4):
                pltpu.make_async_copy(
                    support.at[vidx.at[b, pl.ds(h * 32, 32)]],
                    rows.at[pl.ds(b * CH + h * 32, 32)], sems[b]).wait()

            @pl.when(k + 1 < NCHUNK)
            def _():
                _prep(k + 1, 1 - b)

            pltpu.sync_copy(rows.at[pl.ds(b * CH, CH)],
                            acc.at[uidx.at[b]], add=True)

    plsc.subcore_barrier()
    pltpu.sync_copy(acc.at[pl.ds(sid * RPT, RPT)],
                    pout.at[cid, pl.ds(sid * RPT, RPT)])


def _make_agg():
    mesh = plsc.VectorSubcoreMesh(core_axis_name="c", subcore_axis_name="s",
                                  num_cores=NC, num_subcores=NS)
    return pl.kernel(
        _agg_body,
        out_type=jax.ShapeDtypeStruct((NC, R, D), jnp.float32),
        mesh=mesh,
        scratch_types=[
            pltpu.VMEM((CH + 8,), jnp.int32),
            pltpu.VMEM((CH + 8,), jnp.int32),
            pltpu.VMEM((2, CH), jnp.int32),
            pltpu.VMEM((2, CH), jnp.int32),
            pltpu.VMEM((2 * CH, D), jnp.float32),
            pltpu.VMEM_SHARED((R, D), jnp.float32),
            pltpu.SemaphoreType.DMA,
            pltpu.SemaphoreType.DMA,
        ],
    )


def kernel(input, edge_index, W, b):
    x = input
    n = x.shape[0]
    assert n == N and x.shape[1] == D and edge_index.shape == (2, E)

    support = pl.pallas_call(
        _support_body,
        grid=(N // ROW_BLOCK,),
        in_specs=[pl.BlockSpec((ROW_BLOCK, D), lambda i: (i, 0)),
                  pl.BlockSpec((D, D), lambda i: (0, 0))],
        out_specs=pl.BlockSpec((ROW_BLOCK, D), lambda i: (i, 0)),
        out_shape=jax.ShapeDtypeStruct((N, D), jnp.float32),
    )(x, W)

    enc = edge_index[0].astype(jnp.int32) * N + edge_index[1].astype(jnp.int32)
    skeys = jnp.concatenate([
        jnp.full((8,), -1, jnp.int32),
        jnp.sort(enc),
        jnp.full((E_PAD - E,), N * N, jnp.int32),
    ])
    zrows = jnp.zeros((RPT, D), jnp.float32)

    pout = _make_agg()(skeys, support, zrows)

    out = pl.pallas_call(
        _combine_body,
        grid=(N // ROW_BLOCK,),
        in_specs=[pl.BlockSpec((NC, ROW_BLOCK, D), lambda i: (0, i, 0)),
                  pl.BlockSpec((1, D), lambda i: (0, 0))],
        out_specs=pl.BlockSpec((ROW_BLOCK, D), lambda i: (i, 0)),
        out_shape=jax.ShapeDtypeStruct((N, D), jnp.float32),
    )(pout, b.reshape(1, D))
    return out


# speedup vs baseline: 1.9632x; 1.0249x over previous
import jax
import jax.numpy as jnp
from jax import lax
from jax.experimental import pallas as pl
from jax.experimental.pallas import tpu as pltpu
from jax.experimental.pallas import tpu_sc as plsc

N = 10000
E = 160000
D = 128

NC = 2
NS = 16
L = 16
NW = NC * NS
EPW = 5120
E_PAD = NW * EPW
CH = 128
NCHUNK = EPW // CH
R = 10240
TRASH = N
RPT = R // NS

ROW_BLOCK = 1000


def _support_body(x_ref, w_ref, o_ref):
    d = jnp.dot(x_ref[...], w_ref[...], preferred_element_type=jnp.float32)
    o_ref[...] = d.astype(jnp.bfloat16)


def _combine_body(p_ref, b_ref, o_ref):
    o_ref[...] = p_ref[0] + p_ref[1] + b_ref[...]


def _agg_body(skeys, support, zrows, pout, kv0, kv1, vidx, uidx, rows, frows,
              acc, sem0, sem1):
    cid = lax.axis_index("c")
    sid = lax.axis_index("s")
    wid = sid * NC + cid
    sems = [sem0, sem1]
    kvs = [kv0, kv1]

    pltpu.sync_copy(zrows, acc.at[pl.ds(sid * RPT, RPT)])
    plsc.subcore_barrier()

    base = 8 + wid * EPW

    def _prep(k, b):
        goff = base + k * CH
        pltpu.sync_copy(skeys.at[pl.ds(goff - 8, CH + 8)], kvs[b])
        for i in range(CH // L):
            cur = kvs[b][pl.ds(8 + i * L, L)]
            prv = kvs[b][pl.ds(7 + i * L, L)]
            u = lax.div(cur, N)
            v = cur - u * N
            ue = jnp.where(cur == prv, TRASH, u)
            vidx[b, pl.ds(i * L, L)] = v
            uidx[b, pl.ds(i * L, L)] = ue
        pltpu.async_copy(support.at[vidx.at[b]],
                         rows.at[pl.ds(b * CH, CH)], sems[b])

    _prep(0, 0)

    @pl.loop(0, NCHUNK, step=2)
    def _chunk(c):
        for b in range(2):
            k = c + b
            pltpu.make_async_copy(support.at[vidx.at[b]],
                                  rows.at[pl.ds(b * CH, CH)], sems[b]).wait()

            @pl.when(k + 1 < NCHUNK)
            def _():
                _prep(k + 1, 1 - b)

            @pl.loop(0, CH, step=8)
            def _unpack(r0):
                for dr in range(8):
                    r = r0 + dr
                    for g in range(4):
                        w16 = rows[b * CH + r, pl.ds(g * L, L)]
                        lo = lax.bitcast_convert_type(
                            lax.shift_left(w16, 16), jnp.float32)
                        hi = lax.bitcast_convert_type(
                            w16 & jnp.int32(-65536), jnp.float32)
                        frows[r, pl.ds(2 * g * L, L)] = lo
                        frows[r, pl.ds((2 * g + 1) * L, L)] = hi

            pltpu.sync_copy(frows, acc.at[uidx.at[b]], add=True)

    plsc.subcore_barrier()
    pltpu.sync_copy(acc.at[pl.ds(sid * RPT, RPT)],
                    pout.at[cid, pl.ds(sid * RPT, RPT)])


def _make_agg():
    mesh = plsc.VectorSubcoreMesh(core_axis_name="c", subcore_axis_name="s",
                                  num_cores=NC, num_subcores=NS)
    return pl.kernel(
        _agg_body,
        out_type=jax.ShapeDtypeStruct((NC, R, D), jnp.float32),
        mesh=mesh,
        compiler_params=pltpu.CompilerParams(use_tc_tiling_on_sc=False),
        scratch_types=[
            pltpu.VMEM((CH + 8,), jnp.int32),
            pltpu.VMEM((CH + 8,), jnp.int32),
            pltpu.VMEM((2, CH), jnp.int32),
            pltpu.VMEM((2, CH), jnp.int32),
            pltpu.VMEM((2 * CH, D // 2), jnp.int32),
            pltpu.VMEM((CH, D), jnp.float32),
            pltpu.VMEM_SHARED((R, D), jnp.float32),
            pltpu.SemaphoreType.DMA,
            pltpu.SemaphoreType.DMA,
        ],
    )


def kernel(input, edge_index, W, b):
    x = input
    n = x.shape[0]
    assert n == N and x.shape[1] == D and edge_index.shape == (2, E)

    support = pl.pallas_call(
        _support_body,
        grid=(N // ROW_BLOCK,),
        in_specs=[pl.BlockSpec((ROW_BLOCK, D), lambda i: (i, 0)),
                  pl.BlockSpec((D, D), lambda i: (0, 0))],
        out_specs=pl.BlockSpec((ROW_BLOCK, D), lambda i: (i, 0)),
        out_shape=jax.ShapeDtypeStruct((N, D), jnp.bfloat16),
    )(x, W)
    support = jax.lax.bitcast_convert_type(
        support.reshape(N, D // 2, 2), jnp.int32)

    enc = edge_index[0].astype(jnp.int32) * N + edge_index[1].astype(jnp.int32)
    skeys = jnp.concatenate([
        jnp.full((8,), -1, jnp.int32),
        jnp.sort(enc),
        jnp.full((E_PAD - E,), N * N, jnp.int32),
    ])
    zrows = jnp.zeros((RPT, D), jnp.float32)

    pout = _make_agg()(skeys, support, zrows)

    bp = b.reshape(4, 16, 2).transpose(0, 2, 1).reshape(1, D)
    outp = pl.pallas_call(
        _combine_body,
        grid=(N // ROW_BLOCK,),
        in_specs=[pl.BlockSpec((NC, ROW_BLOCK, D), lambda i: (0, i, 0)),
                  pl.BlockSpec((1, D), lambda i: (0, 0))],
        out_specs=pl.BlockSpec((ROW_BLOCK, D), lambda i: (i, 0)),
        out_shape=jax.ShapeDtypeStruct((N, D), jnp.float32),
    )(pout, bp)
    return outp.reshape(N, 4, 2, 16).transpose(0, 1, 3, 2).reshape(N, D)


# speedup vs baseline: 2.0310x; 1.0346x over previous
import jax
import jax.numpy as jnp
from jax import lax
from jax.experimental import pallas as pl
from jax.experimental.pallas import tpu as pltpu
from jax.experimental.pallas import tpu_sc as plsc

N = 10000
E = 160000
D = 128

NC = 2
NS = 16
L = 16
NW = NC * NS
EPW = 5120
E_PAD = NW * EPW
CH = 128
NCHUNK = EPW // CH
R = 10240
TRASH = N
RPT = R // NS

ROW_BLOCK = 1000


def _support_body(x_ref, w_ref, o_ref):
    d = jnp.dot(x_ref[...], w_ref[...], preferred_element_type=jnp.float32)
    o_ref[...] = d.astype(jnp.bfloat16)


def _combine_body(p_ref, b_ref, o_ref):
    o_ref[...] = p_ref[0] + p_ref[1] + b_ref[...]


def _agg_body(skeys, support, zrows, pout, kv0, kv1, vidx, uidx, rows, frows,
              acc, sem0, sem1, semk0, semk1):
    cid = lax.axis_index("c")
    sid = lax.axis_index("s")
    wid = sid * NC + cid
    sems = [sem0, sem1]
    semk = [semk0, semk1]
    kvs = [kv0, kv1]

    pltpu.sync_copy(zrows, acc.at[pl.ds(sid * RPT, RPT)])
    plsc.subcore_barrier()

    base = 8 + wid * EPW

    def _fire_keys(k, b):
        pltpu.async_copy(skeys.at[pl.ds(base + k * CH - 8, CH + 8)],
                         kvs[b], semk[b])

    def _prep(k, b):
        pltpu.make_async_copy(skeys.at[pl.ds(base + k * CH - 8, CH + 8)],
                              kvs[b], semk[b]).wait()
        for i in range(CH // L):
            cur = kvs[b][pl.ds(8 + i * L, L)]
            prv = kvs[b][pl.ds(7 + i * L, L)]
            u = lax.div(cur, N)
            v = cur - u * N
            ue = jnp.where(cur == prv, TRASH, u)
            vidx[b, pl.ds(i * L, L)] = v
            uidx[b, pl.ds(i * L, L)] = ue
        pltpu.async_copy(support.at[vidx.at[b]],
                         rows.at[pl.ds(b * CH, CH)], sems[b])

        if isinstance(k, int):
            if k + 2 < NCHUNK:
                _fire_keys(k + 2, b)
        else:
            @pl.when(k + 2 < NCHUNK)
            def _():
                _fire_keys(k + 2, b)

    _fire_keys(0, 0)
    _fire_keys(1, 1)
    _prep(0, 0)

    @pl.loop(0, NCHUNK, step=2)
    def _chunk(c):
        for b in range(2):
            k = c + b
            pltpu.make_async_copy(support.at[vidx.at[b]],
                                  rows.at[pl.ds(b * CH, CH)], sems[b]).wait()

            @pl.when(k + 1 < NCHUNK)
            def _():
                _prep(k + 1, 1 - b)

            @pl.loop(0, CH, step=8)
            def _unpack(r0):
                for dr in range(8):
                    r = r0 + dr
                    for g in range(4):
                        w16 = rows[b * CH + r, pl.ds(g * L, L)]
                        lo = lax.bitcast_convert_type(
                            lax.shift_left(w16, 16), jnp.float32)
                        hi = lax.bitcast_convert_type(
                            w16 & jnp.int32(-65536), jnp.float32)
                        frows[r, pl.ds(2 * g * L, L)] = lo
                        frows[r, pl.ds((2 * g + 1) * L, L)] = hi

            pltpu.sync_copy(frows, acc.at[uidx.at[b]], add=True)

    plsc.subcore_barrier()
    pltpu.sync_copy(acc.at[pl.ds(sid * RPT, RPT)],
                    pout.at[cid, pl.ds(sid * RPT, RPT)])


def _make_agg():
    mesh = plsc.VectorSubcoreMesh(core_axis_name="c", subcore_axis_name="s",
                                  num_cores=NC, num_subcores=NS)
    return pl.kernel(
        _agg_body,
        out_type=jax.ShapeDtypeStruct((NC, R, D), jnp.float32),
        mesh=mesh,
        compiler_params=pltpu.CompilerParams(use_tc_tiling_on_sc=False),
        scratch_types=[
            pltpu.VMEM((CH + 8,), jnp.int32),
            pltpu.VMEM((CH + 8,), jnp.int32),
            pltpu.VMEM((2, CH), jnp.int32),
            pltpu.VMEM((2, CH), jnp.int32),
            pltpu.VMEM((2 * CH, D // 2), jnp.int32),
            pltpu.VMEM((CH, D), jnp.float32),
            pltpu.VMEM_SHARED((R, D), jnp.float32),
            pltpu.SemaphoreType.DMA,
            pltpu.SemaphoreType.DMA,
            pltpu.SemaphoreType.DMA,
            pltpu.SemaphoreType.DMA,
        ],
    )


def kernel(input, edge_index, W, b):
    x = input
    n = x.shape[0]
    assert n == N and x.shape[1] == D and edge_index.shape == (2, E)

    support = pl.pallas_call(
        _support_body,
        grid=(N // ROW_BLOCK,),
        in_specs=[pl.BlockSpec((ROW_BLOCK, D), lambda i: (i, 0)),
                  pl.BlockSpec((D, D), lambda i: (0, 0))],
        out_specs=pl.BlockSpec((ROW_BLOCK, D), lambda i: (i, 0)),
        out_shape=jax.ShapeDtypeStruct((N, D), jnp.bfloat16),
    )(x, W)
    support = jax.lax.bitcast_convert_type(
        support.reshape(N, D // 2, 2), jnp.int32)

    enc = edge_index[0].astype(jnp.int32) * N + edge_index[1].astype(jnp.int32)
    skeys = jnp.concatenate([
        jnp.full((8,), -1, jnp.int32),
        jnp.sort(enc),
        jnp.full((E_PAD - E,), N * N, jnp.int32),
    ])
    zrows = jnp.zeros((RPT, D), jnp.float32)

    pout = _make_agg()(skeys, support, zrows)

    bp = b.reshape(4, 16, 2).transpose(0, 2, 1).reshape(1, D)
    outp = pl.pallas_call(
        _combine_body,
        grid=(N // ROW_BLOCK,),
        in_specs=[pl.BlockSpec((NC, ROW_BLOCK, D), lambda i: (0, i, 0)),
                  pl.BlockSpec((1, D), lambda i: (0, 0))],
        out_specs=pl.BlockSpec((ROW_BLOCK, D), lambda i: (i, 0)),
        out_shape=jax.ShapeDtypeStruct((N, D), jnp.float32),
    )(pout, bp)
    return outp.reshape(N, 4, 2, 16).transpose(0, 1, 3, 2).reshape(N, D)
